# Initial kernel scaffold; baseline (speedup 1.0000x reference)
import jax
import jax.numpy as jnp
from jax.experimental import pallas as pl


def kernel(h_new_P, i_embed, edge_index):
    raise NotImplementedError("write your pallas kernel here")


# speedup vs baseline: 1.1062x; 1.1062x over previous
import functools

import jax
import jax.numpy as jnp
from jax import lax
from jax.experimental import pallas as pl
from jax.experimental.pallas import tpu as pltpu
from jax.experimental.pallas import tpu_sc as plsc

N_NODES = 10000
N_EDGES = 320000
D = 128
L = 16
NW = 32
EDGES_PER_W = N_EDGES // NW
C = 80
N_CHUNKS = EDGES_PER_W // C

@functools.lru_cache(maxsize=1)
def _build_score_kernel():
    mesh = plsc.VectorSubcoreMesh(core_axis_name="c", subcore_axis_name="s")

    @functools.partial(
        pl.kernel,
        mesh=mesh,
        compiler_params=pltpu.CompilerParams(needs_layout_passes=False),
        out_type=jax.ShapeDtypeStruct((N_EDGES,), jnp.float32),
        scratch_types=[
            pltpu.VMEM((C,), jnp.int32),
            pltpu.VMEM((C,), jnp.int32),
            pltpu.VMEM((C, D), jnp.float32),
            pltpu.VMEM((C, D), jnp.float32),
            pltpu.VMEM((C,), jnp.float32),
            pltpu.SemaphoreType.DMA,
            pltpu.SemaphoreType.DMA,
        ],
    )
    def _score_kernel(h_hbm, i_hbm, src_hbm, dst_hbm, out_hbm,
                      idx_u, idx_v, u_rows, v_rows, outc, sem_u, sem_v):
        wid = lax.axis_index("s") * 2 + lax.axis_index("c")
        base = wid * EDGES_PER_W

        def chunk_body(k, carry):
            off = base + k * C
            pltpu.sync_copy(src_hbm.at[pl.ds(off, C)], idx_u)
            pltpu.sync_copy(dst_hbm.at[pl.ds(off, C)], idx_v)
            cp_u = pltpu.async_copy(h_hbm.at[idx_u], u_rows, sem_u)
            cp_v = pltpu.async_copy(i_hbm.at[idx_v], v_rows, sem_v)
            cp_u.wait()
            cp_v.wait()

            for e0 in range(0, C, L):
                rows = e0 + lax.iota(jnp.int32, L)

                def d_body(db, acc, rows=rows):
                    for j in range(8):
                        cols = jnp.zeros((L,), jnp.int32) + (db * 8 + j)
                        ug = plsc.load_gather(u_rows, [rows, cols])
                        vg = plsc.load_gather(v_rows, [rows, cols])
                        acc = acc + ug * vg
                    return acc

                acc = lax.fori_loop(0, D // 8, d_body,
                                    jnp.zeros((L,), jnp.float32))
                outc[pl.ds(e0, L)] = acc

            pltpu.sync_copy(outc, out_hbm.at[pl.ds(off, C)])
            return carry

        lax.fori_loop(0, N_CHUNKS, chunk_body, 0)

    return _score_kernel


def kernel(h_new_P, i_embed, edge_index):
    src = edge_index[0].astype(jnp.int32)
    dst = edge_index[1].astype(jnp.int32)
    score = _build_score_kernel()(h_new_P, i_embed, src, dst)
    return score.reshape(N_EDGES, 1)


# speedup vs baseline: 1.3482x; 1.2188x over previous
import functools

import jax
import jax.numpy as jnp
from jax import lax
from jax.experimental import pallas as pl
from jax.experimental.pallas import tpu as pltpu
from jax.experimental.pallas import tpu_sc as plsc

N_NODES = 10000
N_EDGES = 320000
D = 128
L = 16
NW = 32
EDGES_PER_W = N_EDGES // NW
C = 80
N_CHUNKS = EDGES_PER_W // C
N_PAIRS = N_CHUNKS // 2


@functools.lru_cache(maxsize=1)
def _build_score_kernel():
    mesh = plsc.VectorSubcoreMesh(core_axis_name="c", subcore_axis_name="s")

    @functools.partial(
        pl.kernel,
        mesh=mesh,
        compiler_params=pltpu.CompilerParams(needs_layout_passes=False),
        out_type=jax.ShapeDtypeStruct((N_EDGES,), jnp.float32),
        scratch_types=[
            pltpu.VMEM((EDGES_PER_W,), jnp.int32),
            pltpu.VMEM((EDGES_PER_W,), jnp.int32),
            pltpu.VMEM((2, C, D), jnp.float32),
            pltpu.VMEM((2, C, D), jnp.float32),
            pltpu.VMEM((EDGES_PER_W,), jnp.float32),
            pltpu.SemaphoreType.DMA((2,)),
            pltpu.SemaphoreType.DMA((2,)),
        ],
    )
    def _score_kernel(h_hbm, i_hbm, src_hbm, dst_hbm, out_hbm,
                      idx_u, idx_v, u_rows, v_rows, outs, sem_u, sem_v):
        wid = lax.axis_index("s") * 2 + lax.axis_index("c")
        base = wid * EDGES_PER_W
        pltpu.sync_copy(src_hbm.at[pl.ds(base, EDGES_PER_W)], idx_u)
        pltpu.sync_copy(dst_hbm.at[pl.ds(base, EDGES_PER_W)], idx_v)

        def start_gathers(k, b):
            pltpu.async_copy(h_hbm.at[idx_u.at[pl.ds(k * C, C)]],
                             u_rows.at[b], sem_u.at[b])
            pltpu.async_copy(i_hbm.at[idx_v.at[pl.ds(k * C, C)]],
                             v_rows.at[b], sem_v.at[b])

        def wait_gathers(b):
            pltpu.make_async_copy(h_hbm.at[idx_u.at[pl.ds(0, C)]],
                                  u_rows.at[b], sem_u.at[b]).wait()
            pltpu.make_async_copy(i_hbm.at[idx_v.at[pl.ds(0, C)]],
                                  v_rows.at[b], sem_v.at[b]).wait()

        def compute_chunk(k, b):
            ub = u_rows.at[b]
            vb = v_rows.at[b]
            for e0 in range(0, C, L):
                rows = e0 + lax.iota(jnp.int32, L)

                def d_body(db, acc, rows=rows, ub=ub, vb=vb):
                    for j in range(8):
                        cols = jnp.zeros((L,), jnp.int32) + (db * 8 + j)
                        ug = plsc.load_gather(ub, [rows, cols])
                        vg = plsc.load_gather(vb, [rows, cols])
                        acc = acc + ug * vg
                    return acc

                acc = lax.fori_loop(0, D // 8, d_body,
                                    jnp.zeros((L,), jnp.float32))
                outs[pl.ds(k * C + e0, L)] = acc

        start_gathers(0, 0)
        start_gathers(1, 1)

        def pair_body(p, carry):
            k0 = p * 2
            for b in range(2):
                k = k0 + b
                wait_gathers(b)
                compute_chunk(k, b)
                nxt = k + 2

                @pl.when(nxt < N_CHUNKS)
                def _():
                    start_gathers(nxt, b)
            return carry

        lax.fori_loop(0, N_PAIRS, pair_body, 0)

        wait_gathers(0)
        compute_chunk(N_CHUNKS - 1, 0)

        pltpu.sync_copy(outs, out_hbm.at[pl.ds(base, EDGES_PER_W)])

    return _score_kernel


def kernel(h_new_P, i_embed, edge_index):
    src = edge_index[0].astype(jnp.int32)
    dst = edge_index[1].astype(jnp.int32)
    score = _build_score_kernel()(h_new_P, i_embed, src, dst)
    return score.reshape(N_EDGES, 1)


# speedup vs baseline: 9.1058x; 6.7541x over previous
import functools

import jax
import jax.numpy as jnp
from jax import lax
from jax.experimental import pallas as pl
from jax.experimental.pallas import tpu as pltpu
from jax.experimental.pallas import tpu_sc as plsc

N_NODES = 10000
N_EDGES = 320000
D = 128
L = 16
NW = 32
EDGES_PER_W = N_EDGES // NW
C = 80
N_CHUNKS = EDGES_PER_W // C
N_PAIRS = N_CHUNKS // 2


@functools.lru_cache(maxsize=1)
def _build_score_kernel():
    mesh = plsc.VectorSubcoreMesh(core_axis_name="c", subcore_axis_name="s")

    @functools.partial(
        pl.kernel,
        mesh=mesh,
        compiler_params=pltpu.CompilerParams(needs_layout_passes=False),
        out_type=jax.ShapeDtypeStruct((N_EDGES,), jnp.float32),
        scratch_types=[
            pltpu.VMEM((EDGES_PER_W,), jnp.int32),
            pltpu.VMEM((EDGES_PER_W,), jnp.int32),
            pltpu.VMEM((2, C, D), jnp.float32),
            pltpu.VMEM((2, C, D), jnp.float32),
            pltpu.VMEM((EDGES_PER_W,), jnp.float32),
            pltpu.SemaphoreType.DMA((2,)),
            pltpu.SemaphoreType.DMA((2,)),
        ],
    )
    def _score_kernel(h_hbm, i_hbm, src_hbm, dst_hbm, out_hbm,
                      idx_u, idx_v, u_rows, v_rows, outs, sem_u, sem_v):
        wid = lax.axis_index("s") * 2 + lax.axis_index("c")
        base = wid * EDGES_PER_W
        pltpu.sync_copy(src_hbm.at[pl.ds(base, EDGES_PER_W)], idx_u)
        pltpu.sync_copy(dst_hbm.at[pl.ds(base, EDGES_PER_W)], idx_v)

        def start_gathers(k, b):
            pltpu.async_copy(h_hbm.at[idx_u.at[pl.ds(k * C, C)]],
                             u_rows.at[b], sem_u.at[b])
            pltpu.async_copy(i_hbm.at[idx_v.at[pl.ds(k * C, C)]],
                             v_rows.at[b], sem_v.at[b])

        def wait_gathers(b):
            pltpu.make_async_copy(h_hbm.at[idx_u.at[pl.ds(0, C)]],
                                  u_rows.at[b], sem_u.at[b]).wait()
            pltpu.make_async_copy(i_hbm.at[idx_v.at[pl.ds(0, C)]],
                                  v_rows.at[b], sem_v.at[b]).wait()

        def compute_chunk(k, b):
            ub = u_rows.at[b]
            vb = v_rows.at[b]
            lanes = lax.iota(jnp.int32, L)
            for e0 in range(0, C, L):
                rows = e0 + lanes

                def d_body(db, acc, rows=rows, ub=ub, vb=vb):
                    for j in range(8):
                        cols = (lanes + (db * 8 + j)) & (D - 1)
                        ug = plsc.load_gather(ub, [rows, cols])
                        vg = plsc.load_gather(vb, [rows, cols])
                        acc = acc + ug * vg
                    return acc

                acc = lax.fori_loop(0, D // 8, d_body,
                                    jnp.zeros((L,), jnp.float32))
                outs[pl.ds(k * C + e0, L)] = acc

        start_gathers(0, 0)
        start_gathers(1, 1)

        def pair_body(p, carry):
            k0 = p * 2
            for b in range(2):
                k = k0 + b
                wait_gathers(b)
                compute_chunk(k, b)
                nxt = k + 2

                @pl.when(nxt < N_CHUNKS)
                def _():
                    start_gathers(nxt, b)
            return carry

        lax.fori_loop(0, N_PAIRS, pair_body, 0)

        wait_gathers(0)
        compute_chunk(N_CHUNKS - 1, 0)

        pltpu.sync_copy(outs, out_hbm.at[pl.ds(base, EDGES_PER_W)])

    return _score_kernel


def kernel(h_new_P, i_embed, edge_index):
    src = edge_index[0].astype(jnp.int32)
    dst = edge_index[1].astype(jnp.int32)
    score = _build_score_kernel()(h_new_P, i_embed, src, dst)
    return score.reshape(N_EDGES, 1)
